# baseline (speedup 1.0000x reference)
import dataclasses
import functools

import jax
import jax.numpy as jnp
from jax import lax
from jax.experimental import pallas as pl
from jax.experimental.pallas import tpu as pltpu
from jax.experimental.pallas import tpu_sc as plsc

_NW = 32
_L = 16


def _sc_body(N, V, MW, G, CH1, CH2,
             in_hbm, tgt_hbm, msk_hbm, mif_hbm, mtf_hbm, mmf_hbm, mmt_hbm,
             out_hbm,
             tgt_v, msk_v, idx1_v, val1_v, mt_v, mm_v, idx2_v, val2_v,
             mmt_v, res_v, sem):
    wid = lax.axis_index("s") * 2 + lax.axis_index("c")
    iota = lax.iota(jnp.int32, _L)

    b1 = wid * CH1
    pltpu.sync_copy(tgt_hbm.at[pl.ds(b1, CH1)], tgt_v)
    pltpu.sync_copy(msk_hbm.at[pl.ds(b1, CH1)], msk_v)
    for c in range(CH1 // _L):
        j = b1 + (c * _L) + iota
        row = jnp.minimum(j, N - 1)
        idx1_v[pl.ds(c * _L, _L)] = row * V + tgt_v[pl.ds(c * _L, _L)]
    pltpu.async_copy(in_hbm.at[idx1_v], val1_v, sem).wait()

    b2 = wid * CH2
    pltpu.sync_copy(mtf_hbm.at[pl.ds(b2, CH2)], mt_v)
    pltpu.sync_copy(mmf_hbm.at[pl.ds(b2, CH2)], mm_v)
    for c in range(CH2 // _L):
        j = b2 + (c * _L) + iota
        row = jnp.minimum(lax.shift_right_logical(j, 2), N - 1)
        mt = mt_v[pl.ds(c * _L, _L)]
        idx2_v[pl.ds(c * _L, _L)] = jnp.maximum(row * MW + mt - 1, 0)
    for g in range(CH2 // CH1):
        pltpu.async_copy(
            mif_hbm.at[idx2_v.at[pl.ds(g * CH1, CH1)]],
            val2_v.at[pl.ds(g * CH1, CH1)], sem).wait()

    for g in range(G):
        pltpu.sync_copy(mmt_hbm.at[pl.ds(g * (_NW * CH1) + b1, CH1)],
                        mmt_v.at[pl.ds(g * CH1, CH1)])

    zero = jnp.zeros((_L,), jnp.float32)
    acc1 = zero
    accm = zero
    for c in range(CH1 // _L):
        m = msk_v[pl.ds(c * _L, _L)]
        acc1 = acc1 + val1_v[pl.ds(c * _L, _L)] * m
        accm = accm + m
    acc2 = zero
    for c in range(CH2 // _L):
        mt = mt_v[pl.ds(c * _L, _L)]
        mm = mm_v[pl.ds(c * _L, _L)]
        acc2 = acc2 + val2_v[pl.ds(c * _L, _L)] * jnp.where(mt == 0, 0.0, mm)
    cnt = zero
    for c in range(CH1 // _L):
        rs = zero
        for g in range(G):
            rs = rs + mmt_v[pl.ds(g * CH1 + c * _L, _L)]
        cnt = cnt + jnp.where(rs != 0.0, 1.0, 0.0)

    s1 = jnp.sum(acc1)
    sm = jnp.sum(accm)
    s2 = jnp.sum(acc2)
    sc = jnp.sum(cnt)
    res_v[...] = (jnp.where(iota == 0, -s1, 0.0)
                  + jnp.where(iota == 1, sm, 0.0)
                  + jnp.where(iota == 2, -s2, 0.0)
                  + jnp.where(iota == 3, sc, 0.0))
    pltpu.sync_copy(res_v, out_hbm.at[wid])


def kernel(input, target, mask, match_input, match_target, match_mask):
    B, S, V = input.shape
    MW = match_input.shape[2]
    G = match_target.shape[2]
    N = B * S

    CH1 = ((N + _NW - 1) // _NW + _L - 1) // _L * _L
    CH2 = ((N * G + _NW - 1) // _NW + _L - 1) // _L * _L
    CH2 = (CH2 + CH1 - 1) // CH1 * CH1
    P1 = _NW * CH1
    P2 = _NW * CH2

    f32 = jnp.float32
    i32 = jnp.int32
    inf = input.reshape(-1)
    tgt = jnp.pad(target.reshape(-1).astype(i32), (0, P1 - N))
    msk = jnp.pad(mask.reshape(-1).astype(f32), (0, P1 - N))
    mif = match_input.reshape(-1)
    mtf = jnp.pad(match_target.reshape(-1).astype(i32), (0, P2 - N * G))
    mmf = jnp.pad(match_mask.reshape(-1).astype(f32), (0, P2 - N * G))
    mmt = jnp.pad(match_mask.reshape(N, G).astype(f32).T,
                  ((0, 0), (0, P1 - N))).reshape(-1)

    mesh = plsc.VectorSubcoreMesh(core_axis_name="c", subcore_axis_name="s")
    body = functools.partial(_sc_body, N, V, MW, G, CH1, CH2)
    cp = pltpu.CompilerParams()
    if "needs_layout_passes" in pltpu.CompilerParams.__dataclass_fields__:
        cp = dataclasses.replace(cp, needs_layout_passes=False)
    out = pl.kernel(
        body,
        out_type=jax.ShapeDtypeStruct((_NW, _L), f32),
        mesh=mesh,
        compiler_params=cp,
        scratch_types=[
            pltpu.VMEM((CH1,), i32),
            pltpu.VMEM((CH1,), f32),
            pltpu.VMEM((CH1,), i32),
            pltpu.VMEM((CH1,), f32),
            pltpu.VMEM((CH2,), i32),
            pltpu.VMEM((CH2,), f32),
            pltpu.VMEM((CH2,), i32),
            pltpu.VMEM((CH2,), f32),
            pltpu.VMEM((G * CH1,), f32),
            pltpu.VMEM((_L,), f32),
            pltpu.SemaphoreType.DMA,
        ],
    )(inf, tgt, msk, mif, mtf, mmf, mmt)

    p = out.sum(axis=0)
    return (p[0] / p[1], p[2] / p[3])


# speedup vs baseline: 10.0761x; 10.0761x over previous
import dataclasses
import functools

import jax
import jax.numpy as jnp
from jax import lax
from jax.experimental import pallas as pl
from jax.experimental.pallas import tpu as pltpu
from jax.experimental.pallas import tpu_sc as plsc

_NW = 32
_L = 16


def _tc_nll_body(S, V, x_ref, t_ref, m_ref, out_ref):
    @pl.when(pl.program_id(0) == 0)
    def _():
        out_ref[...] = jnp.zeros_like(out_ref)

    x = x_ref[0]
    t = t_ref[0, 0]
    m = m_ref[0, 0]
    col = lax.broadcasted_iota(jnp.int32, (S, V), 1)
    sel = jnp.where(col == t[:, None], x, 0.0)
    nll_blk = jnp.sum(jnp.sum(sel, axis=1) * m)
    msk_blk = jnp.sum(m)
    r8 = lax.broadcasted_iota(jnp.int32, (8, 128), 0)
    c128 = lax.broadcasted_iota(jnp.int32, (8, 128), 1)
    out_ref[...] += jnp.where((r8 == 0) & (c128 == 0), nll_blk, 0.0) \
        + jnp.where((r8 == 0) & (c128 == 1), msk_blk, 0.0)


def _sc_match_body(N, MW, G, CH1, CH2,
                   mif_hbm, mtf_hbm, mmf_hbm, mmt_hbm, out_hbm,
                   mt_v, mm_v, idx2_v, val2_v, mmt_v, res_v, sem):
    wid = lax.axis_index("s") * 2 + lax.axis_index("c")
    iota = lax.iota(jnp.int32, _L)

    b1 = wid * CH1
    b2 = wid * CH2
    pltpu.sync_copy(mtf_hbm.at[pl.ds(b2, CH2)], mt_v)
    pltpu.sync_copy(mmf_hbm.at[pl.ds(b2, CH2)], mm_v)
    for c in range(CH2 // _L):
        j = b2 + (c * _L) + iota
        row = jnp.minimum(lax.shift_right_logical(j, 2), N - 1)
        mt = mt_v[pl.ds(c * _L, _L)]
        idx2_v[pl.ds(c * _L, _L)] = jnp.maximum(row * MW + mt - 1, 0)
    for g in range(CH2 // CH1):
        pltpu.async_copy(
            mif_hbm.at[idx2_v.at[pl.ds(g * CH1, CH1)]],
            val2_v.at[pl.ds(g * CH1, CH1)], sem).wait()

    for g in range(G):
        pltpu.sync_copy(mmt_hbm.at[pl.ds(g * (_NW * CH1) + b1, CH1)],
                        mmt_v.at[pl.ds(g * CH1, CH1)])

    zero = jnp.zeros((_L,), jnp.float32)
    acc2 = zero
    for c in range(CH2 // _L):
        mt = mt_v[pl.ds(c * _L, _L)]
        mm = mm_v[pl.ds(c * _L, _L)]
        acc2 = acc2 + val2_v[pl.ds(c * _L, _L)] * jnp.where(mt == 0, 0.0, mm)
    cnt = zero
    for c in range(CH1 // _L):
        rs = zero
        for g in range(G):
            rs = rs + mmt_v[pl.ds(g * CH1 + c * _L, _L)]
        cnt = cnt + jnp.where(rs != 0.0, 1.0, 0.0)

    s2 = jnp.sum(acc2)
    sc = jnp.sum(cnt)
    res_v[...] = jnp.where(iota == 0, -s2, 0.0) + jnp.where(iota == 1, sc, 0.0)
    pltpu.sync_copy(res_v, out_hbm.at[wid])


def kernel(input, target, mask, match_input, match_target, match_mask):
    B, S, V = input.shape
    MW = match_input.shape[2]
    G = match_target.shape[2]
    N = B * S

    f32 = jnp.float32
    i32 = jnp.int32
    tgt3 = target.astype(i32).reshape(B, 1, S)
    msk3 = mask.astype(f32).reshape(B, 1, S)

    nll = pl.pallas_call(
        functools.partial(_tc_nll_body, S, V),
        grid=(B,),
        in_specs=[
            pl.BlockSpec((1, S, V), lambda i: (i, 0, 0)),
            pl.BlockSpec((1, 1, S), lambda i: (i, 0, 0)),
            pl.BlockSpec((1, 1, S), lambda i: (i, 0, 0)),
        ],
        out_specs=pl.BlockSpec((8, 128), lambda i: (0, 0)),
        out_shape=jax.ShapeDtypeStruct((8, 128), f32),
    )(input, tgt3, msk3)

    CH1 = ((N + _NW - 1) // _NW + _L - 1) // _L * _L
    CH2 = ((N * G + _NW - 1) // _NW + _L - 1) // _L * _L
    CH2 = (CH2 + CH1 - 1) // CH1 * CH1
    P1 = _NW * CH1
    P2 = _NW * CH2

    mif = match_input.reshape(-1)
    mtf = jnp.pad(match_target.reshape(-1).astype(i32), (0, P2 - N * G))
    mmf = jnp.pad(match_mask.reshape(-1).astype(f32), (0, P2 - N * G))
    mmt = jnp.pad(match_mask.reshape(N, G).astype(f32).T,
                  ((0, 0), (0, P1 - N))).reshape(-1)

    mesh = plsc.VectorSubcoreMesh(core_axis_name="c", subcore_axis_name="s")
    body = functools.partial(_sc_match_body, N, MW, G, CH1, CH2)
    cp = pltpu.CompilerParams()
    if "needs_layout_passes" in pltpu.CompilerParams.__dataclass_fields__:
        cp = dataclasses.replace(cp, needs_layout_passes=False)
    out = pl.kernel(
        body,
        out_type=jax.ShapeDtypeStruct((_NW, _L), f32),
        mesh=mesh,
        compiler_params=cp,
        scratch_types=[
            pltpu.VMEM((CH2,), i32),
            pltpu.VMEM((CH2,), f32),
            pltpu.VMEM((CH2,), i32),
            pltpu.VMEM((CH2,), f32),
            pltpu.VMEM((G * CH1,), f32),
            pltpu.VMEM((_L,), f32),
            pltpu.SemaphoreType.DMA,
        ],
    )(mif, mtf, mmf, mmt)

    p = out.sum(axis=0)
    return (-nll[0, 0] / nll[0, 1], p[0] / p[1])


# speedup vs baseline: 19.9013x; 1.9751x over previous
import dataclasses
import functools

import jax
import jax.numpy as jnp
from jax import lax
from jax.experimental import pallas as pl
from jax.experimental.pallas import tpu as pltpu
from jax.experimental.pallas import tpu_sc as plsc

_NW = 32
_L = 16


def _tc_nll_body(SB, B, V, x_ref, t_ref, m_ref, out_ref):
    @pl.when(pl.program_id(0) == 0)
    def _():
        out_ref[...] = jnp.zeros_like(out_ref)

    x = x_ref[...]
    t = t_ref[0]
    m = m_ref[0]
    col = lax.broadcasted_iota(jnp.int32, (SB, B, V), 2)
    sel = jnp.where(col == t[:, :, None], x, 0.0)
    nll_blk = jnp.sum(jnp.sum(sel, axis=2) * m)
    msk_blk = jnp.sum(m)
    r8 = lax.broadcasted_iota(jnp.int32, (8, 128), 0)
    c128 = lax.broadcasted_iota(jnp.int32, (8, 128), 1)
    out_ref[...] += jnp.where((r8 == 0) & (c128 == 0), nll_blk, 0.0) \
        + jnp.where((r8 == 0) & (c128 == 1), msk_blk, 0.0)


def _sc_match_body(N, MW, G, CH1, CH2,
                   mif_hbm, mtf_hbm, mmf_hbm, mmt_hbm, out_hbm,
                   mt_v, mm_v, idx2_v, val2_v, mmt_v, res_v, sem):
    wid = lax.axis_index("s") * 2 + lax.axis_index("c")
    iota = lax.iota(jnp.int32, _L)

    b1 = wid * CH1
    b2 = wid * CH2
    pltpu.sync_copy(mtf_hbm.at[pl.ds(b2, CH2)], mt_v)
    pltpu.sync_copy(mmf_hbm.at[pl.ds(b2, CH2)], mm_v)
    for c in range(CH2 // _L):
        j = b2 + (c * _L) + iota
        row = jnp.minimum(lax.shift_right_logical(j, 2), N - 1)
        mt = mt_v[pl.ds(c * _L, _L)]
        idx2_v[pl.ds(c * _L, _L)] = jnp.maximum(row * MW + mt - 1, 0)
    for g in range(CH2 // CH1):
        pltpu.async_copy(
            mif_hbm.at[idx2_v.at[pl.ds(g * CH1, CH1)]],
            val2_v.at[pl.ds(g * CH1, CH1)], sem).wait()

    for g in range(G):
        pltpu.sync_copy(mmt_hbm.at[pl.ds(g * (_NW * CH1) + b1, CH1)],
                        mmt_v.at[pl.ds(g * CH1, CH1)])

    zero = jnp.zeros((_L,), jnp.float32)
    acc2 = zero
    for c in range(CH2 // _L):
        mt = mt_v[pl.ds(c * _L, _L)]
        mm = mm_v[pl.ds(c * _L, _L)]
        acc2 = acc2 + val2_v[pl.ds(c * _L, _L)] * jnp.where(mt == 0, 0.0, mm)
    cnt = zero
    for c in range(CH1 // _L):
        rs = zero
        for g in range(G):
            rs = rs + mmt_v[pl.ds(g * CH1 + c * _L, _L)]
        cnt = cnt + jnp.where(rs != 0.0, 1.0, 0.0)

    s2 = jnp.sum(acc2)
    sc = jnp.sum(cnt)
    res_v[...] = jnp.where(iota == 0, -s2, 0.0) + jnp.where(iota == 1, sc, 0.0)
    pltpu.sync_copy(res_v, out_hbm.at[wid])


def kernel(input, target, mask, match_input, match_target, match_mask):
    B, S, V = input.shape
    MW = match_input.shape[2]
    G = match_target.shape[2]
    N = B * S

    f32 = jnp.float32
    i32 = jnp.int32
    xt = jnp.transpose(input, (1, 0, 2))
    SB = 4
    tgt3 = target.astype(i32).T.reshape(S // SB, SB, B)
    msk3 = mask.astype(f32).T.reshape(S // SB, SB, B)

    nll = pl.pallas_call(
        functools.partial(_tc_nll_body, SB, B, V),
        grid=(S // SB,),
        in_specs=[
            pl.BlockSpec((SB, B, V), lambda i: (i, 0, 0)),
            pl.BlockSpec((1, SB, B), lambda i: (i, 0, 0)),
            pl.BlockSpec((1, SB, B), lambda i: (i, 0, 0)),
        ],
        out_specs=pl.BlockSpec((8, 128), lambda i: (0, 0)),
        out_shape=jax.ShapeDtypeStruct((8, 128), f32),
    )(xt, tgt3, msk3)

    CH1 = ((N + _NW - 1) // _NW + _L - 1) // _L * _L
    CH2 = ((N * G + _NW - 1) // _NW + _L - 1) // _L * _L
    CH2 = (CH2 + CH1 - 1) // CH1 * CH1
    P1 = _NW * CH1
    P2 = _NW * CH2

    mif = match_input.reshape(-1)
    mtf = jnp.pad(match_target.reshape(-1).astype(i32), (0, P2 - N * G))
    mmf = jnp.pad(match_mask.reshape(-1).astype(f32), (0, P2 - N * G))
    mmt = jnp.pad(match_mask.reshape(N, G).astype(f32).T,
                  ((0, 0), (0, P1 - N))).reshape(-1)

    mesh = plsc.VectorSubcoreMesh(core_axis_name="c", subcore_axis_name="s")
    body = functools.partial(_sc_match_body, N, MW, G, CH1, CH2)
    cp = pltpu.CompilerParams()
    if "needs_layout_passes" in pltpu.CompilerParams.__dataclass_fields__:
        cp = dataclasses.replace(cp, needs_layout_passes=False)
    out = pl.kernel(
        body,
        out_type=jax.ShapeDtypeStruct((_NW, _L), f32),
        mesh=mesh,
        compiler_params=cp,
        scratch_types=[
            pltpu.VMEM((CH2,), i32),
            pltpu.VMEM((CH2,), f32),
            pltpu.VMEM((CH2,), i32),
            pltpu.VMEM((CH2,), f32),
            pltpu.VMEM((G * CH1,), f32),
            pltpu.VMEM((_L,), f32),
            pltpu.SemaphoreType.DMA,
        ],
    )(mif, mtf, mmf, mmt)

    p = out.sum(axis=0)
    return (-nll[0, 0] / nll[0, 1], p[0] / p[1])


# speedup vs baseline: 23.6831x; 1.1900x over previous
import dataclasses
import functools

import jax
import jax.numpy as jnp
from jax import lax
from jax.experimental import pallas as pl
from jax.experimental.pallas import tpu as pltpu
from jax.experimental.pallas import tpu_sc as plsc

_NW = 32
_L = 16


def _tc_nll_body(SB, B, V, x_ref, t_ref, m_ref, out_ref):
    i = pl.program_id(0)

    @pl.when(i == 0)
    def _():
        out_ref[...] = jnp.zeros_like(out_ref)

    x = x_ref[...]
    t = t_ref[pl.ds(i * SB, SB), :]
    m = m_ref[pl.ds(i * SB, SB), :]
    col = lax.broadcasted_iota(jnp.int32, (SB, B, V), 2)
    sel = jnp.where(col == t[:, :, None], x, 0.0)
    nll_blk = jnp.sum(jnp.sum(sel, axis=2) * m)
    msk_blk = jnp.sum(m)
    r8 = lax.broadcasted_iota(jnp.int32, (8, 128), 0)
    c128 = lax.broadcasted_iota(jnp.int32, (8, 128), 1)
    out_ref[...] += jnp.where((r8 == 0) & (c128 == 0), nll_blk, 0.0) \
        + jnp.where((r8 == 0) & (c128 == 1), msk_blk, 0.0)


def _sc_match_body(N, MW, G, CH2, CHR,
                   mif_hbm, mtf_hbm, mmf_hbm, out_hbm,
                   mt_v, mm_v, idx2_v, val2_v, res_v, sem):
    wid = lax.axis_index("s") * 2 + lax.axis_index("c")
    iota = lax.iota(jnp.int32, _L)
    zero = jnp.zeros((_L,), jnp.float32)

    b2 = wid * CH2
    pltpu.sync_copy(mtf_hbm.at[pl.ds(b2, CH2)], mt_v)
    pltpu.sync_copy(mmf_hbm.at[pl.ds(b2, CH2)], mm_v.at[pl.ds(0, CH2)])
    for c in range(CH2 // _L, (G * CHR) // _L):
        mm_v[pl.ds(c * _L, _L)] = zero
    for c in range(CH2 // _L):
        j = b2 + (c * _L) + iota
        row = jnp.minimum(lax.shift_right_logical(j, 2), N - 1)
        mt = mt_v[pl.ds(c * _L, _L)]
        idx2_v[pl.ds(c * _L, _L)] = jnp.maximum(row * MW + mt - 1, 0)
    splits = list(range(0, CH2, 128)) + [CH2]
    cps = []
    for lo, hi in zip(splits[:-1], splits[1:]):
        cps.append(pltpu.async_copy(
            mif_hbm.at[idx2_v.at[pl.ds(lo, hi - lo)]],
            val2_v.at[pl.ds(lo, hi - lo)], sem))
    for cp in cps:
        cp.wait()

    acc2 = zero
    for c in range(CH2 // _L):
        mt = mt_v[pl.ds(c * _L, _L)]
        mm = mm_v[pl.ds(c * _L, _L)]
        acc2 = acc2 + val2_v[pl.ds(c * _L, _L)] * jnp.where(mt == 0, 0.0, mm)
    cnt = zero
    for c in range(CHR // _L):
        base = c * _L * G
        rs = zero
        for g in range(G):
            rs = rs + plsc.load_gather(mm_v, [iota * G + (base + g)])
        cnt = cnt + jnp.where(rs != 0.0, 1.0, 0.0)

    s2 = jnp.sum(acc2)
    sc = jnp.sum(cnt)
    res_v[...] = jnp.where(iota == 0, -s2, 0.0) + jnp.where(iota == 1, sc, 0.0)
    pltpu.sync_copy(res_v, out_hbm.at[wid])


def kernel(input, target, mask, match_input, match_target, match_mask):
    B, S, V = input.shape
    MW = match_input.shape[2]
    G = match_target.shape[2]
    N = B * S

    f32 = jnp.float32
    i32 = jnp.int32
    xt = jnp.transpose(input, (1, 0, 2))
    SB = 10
    tgt_t = target.astype(i32).T
    msk_t = mask.astype(f32).T

    nll = pl.pallas_call(
        functools.partial(_tc_nll_body, SB, B, V),
        grid=(S // SB,),
        in_specs=[
            pl.BlockSpec((SB, B, V), lambda i: (i, 0, 0)),
            pl.BlockSpec((S, B), lambda i: (0, 0)),
            pl.BlockSpec((S, B), lambda i: (0, 0)),
        ],
        out_specs=pl.BlockSpec((8, 128), lambda i: (0, 0)),
        out_shape=jax.ShapeDtypeStruct((8, 128), f32),
    )(xt, tgt_t, msk_t)

    CH2 = N * G // _NW
    CHR = N // _NW
    CHR_PAD = (CHR + _L - 1) // _L * _L

    mif = match_input.reshape(-1)
    mtf = match_target.reshape(-1).astype(i32)
    mmf = match_mask.reshape(-1).astype(f32)

    mesh = plsc.VectorSubcoreMesh(core_axis_name="c", subcore_axis_name="s")
    body = functools.partial(_sc_match_body, N, MW, G, CH2, CHR_PAD)
    cp = pltpu.CompilerParams()
    if "needs_layout_passes" in pltpu.CompilerParams.__dataclass_fields__:
        cp = dataclasses.replace(cp, needs_layout_passes=False)
    out = pl.kernel(
        body,
        out_type=jax.ShapeDtypeStruct((_NW, _L), f32),
        mesh=mesh,
        compiler_params=cp,
        scratch_types=[
            pltpu.VMEM((CH2,), i32),
            pltpu.VMEM((G * CHR_PAD,), f32),
            pltpu.VMEM((CH2,), i32),
            pltpu.VMEM((CH2,), f32),
            pltpu.VMEM((_L,), f32),
            pltpu.SemaphoreType.DMA,
        ],
    )(mif, mtf, mmf)

    p = out.sum(axis=0)
    return (-nll[0, 0] / nll[0, 1], p[0] / p[1])
